# baseline (speedup 1.0000x reference)
import functools

import jax
import jax.numpy as jnp
from jax.experimental import pallas as pl
from jax.experimental.pallas import tpu as pltpu


def _segsum_kernel(num_edges, tiles_per_slab, mask_tail,
                   et_ref, c_ref, ones_ref, sums_ref, cnt_ref):
    e = pl.program_id(1)

    @pl.when(e == 0)
    def _init():
        sums_ref[...] = jnp.zeros_like(sums_ref)
        cnt_ref[...] = jnp.zeros_like(cnt_ref)

    num_rel = sums_ref.shape[0]
    tile_e = c_ref.shape[0]
    rel = jax.lax.broadcasted_iota(jnp.int32, (num_rel, tile_e), 0)
    onehot = (rel == et_ref[...]).astype(jnp.float32)

    if mask_tail:
        tile_idx = pl.program_id(0) * tiles_per_slab + e
        rows = tile_idx * tile_e + jax.lax.broadcasted_iota(
            jnp.int32, (tile_e, 1), 0)
        c_tile = jnp.where(rows < num_edges, c_ref[...], 0.0)
    else:
        c_tile = c_ref[...]

    sums_ref[...] += jnp.dot(onehot, c_tile,
                             preferred_element_type=jnp.float32)
    cnt_ref[...] += jnp.dot(onehot, ones_ref[...],
                            preferred_element_type=jnp.float32)


def _finish_kernel(sums_ref, cnt_ref, g_ref, wg_ref, wrel_ref, brel_ref,
                   out_ref):
    sums = jnp.sum(sums_ref[...], axis=0)
    counts = jnp.sum(cnt_ref[...], axis=0)[:, :1]
    means = sums / jnp.maximum(counts, 1.0)
    out_ref[...] = (
        jnp.dot(means, wg_ref[...], preferred_element_type=jnp.float32)
        + jnp.dot(g_ref[...], wrel_ref[...],
                  preferred_element_type=jnp.float32)
        + brel_ref[...])


def kernel(g_initial, c_ijk, edge_type, wg_t, wrel_t, brel):
    R, in_size = g_initial.shape
    E, c3 = c_ijk.shape
    out_size = wrel_t.shape[1]
    f32 = jnp.float32

    tile_e = max(128, (min(8192, E) // 128) * 128)
    c = c_ijk
    if E < tile_e:
        c = jnp.pad(c, ((0, tile_e - E), (0, 0)))

    num_tiles = pl.cdiv(E, tile_e)
    num_slabs = 2 if num_tiles > 1 else 1
    tiles_per_slab = pl.cdiv(num_tiles, num_slabs)
    last_tile = num_tiles - 1

    et_len = num_slabs * tiles_per_slab * tile_e
    et_p = jnp.full((1, et_len), -1, jnp.int32)
    et_p = et_p.at[0, :E].set(edge_type.astype(jnp.int32))
    ones = jnp.ones((tile_e, 128), f32)

    sums, cnt = pl.pallas_call(
        functools.partial(_segsum_kernel, E, tiles_per_slab,
                          bool(E % tile_e)),
        out_shape=(
            jax.ShapeDtypeStruct((num_slabs, R, c3), f32),
            jax.ShapeDtypeStruct((num_slabs, R, 128), f32),
        ),
        grid=(num_slabs, tiles_per_slab),
        in_specs=[
            pl.BlockSpec((1, tile_e),
                         lambda s, e: (0, s * tiles_per_slab + e)),
            pl.BlockSpec((tile_e, c3),
                         lambda s, e: (jnp.minimum(
                             s * tiles_per_slab + e, last_tile), 0)),
            pl.BlockSpec((tile_e, 128), lambda s, e: (0, 0)),
        ],
        out_specs=(
            pl.BlockSpec((None, R, c3), lambda s, e: (s, 0, 0)),
            pl.BlockSpec((None, R, 128), lambda s, e: (s, 0, 0)),
        ),
        compiler_params=pltpu.CompilerParams(
            dimension_semantics=("parallel", "arbitrary"),
            vmem_limit_bytes=64 * 1024 * 1024,
        ),
    )(et_p, c, ones)

    out = pl.pallas_call(
        _finish_kernel,
        out_shape=jax.ShapeDtypeStruct((R, out_size), f32),
    )(sums, cnt, g_initial.astype(f32), wg_t.astype(f32),
      wrel_t.astype(f32), brel.astype(f32))
    return out


# speedup vs baseline: 1.1249x; 1.1249x over previous
import functools

import jax
import jax.numpy as jnp
from jax.experimental import pallas as pl
from jax.experimental.pallas import tpu as pltpu


def _segsum_kernel(num_edges, tiles_per_slab, mask_tail,
                   et_ref, c_ref, sums_ref, cnt_ref):
    e = pl.program_id(1)

    @pl.when(e == 0)
    def _init():
        sums_ref[...] = jnp.zeros_like(sums_ref)
        cnt_ref[...] = jnp.zeros_like(cnt_ref)

    num_rel = sums_ref.shape[0]
    tile_e = c_ref.shape[0]
    rel = jax.lax.broadcasted_iota(jnp.int32, (num_rel, tile_e), 0)
    onehot = (rel == et_ref[...]).astype(jnp.float32)

    if mask_tail:
        tile_idx = pl.program_id(0) * tiles_per_slab + e
        rows = tile_idx * tile_e + jax.lax.broadcasted_iota(
            jnp.int32, (tile_e, 1), 0)
        c_tile = jnp.where(rows < num_edges, c_ref[...], 0.0)
    else:
        c_tile = c_ref[...]

    sums_ref[...] += jnp.dot(onehot, c_tile,
                             preferred_element_type=jnp.float32)
    cnt_ref[...] += jnp.sum(onehot, axis=1, keepdims=True)


def _finish_kernel(sums_ref, cnt_ref, g_ref, wg_ref, wrel_ref, brel_ref,
                   out_ref):
    sums = jnp.sum(sums_ref[...], axis=0)
    counts = jnp.sum(cnt_ref[...], axis=0)
    means = sums / jnp.maximum(counts, 1.0)
    out_ref[...] = (
        jnp.dot(means, wg_ref[...], preferred_element_type=jnp.float32)
        + jnp.dot(g_ref[...], wrel_ref[...],
                  preferred_element_type=jnp.float32)
        + brel_ref[...])


def kernel(g_initial, c_ijk, edge_type, wg_t, wrel_t, brel):
    R, in_size = g_initial.shape
    E, c3 = c_ijk.shape
    out_size = wrel_t.shape[1]
    f32 = jnp.float32

    tile_e = max(128, (min(8192, E) // 128) * 128)
    c = c_ijk
    if E < tile_e:
        c = jnp.pad(c, ((0, tile_e - E), (0, 0)))

    num_tiles = pl.cdiv(E, tile_e)
    num_slabs = 2 if num_tiles > 1 else 1
    tiles_per_slab = pl.cdiv(num_tiles, num_slabs)
    last_tile = num_tiles - 1

    et_len = num_slabs * tiles_per_slab * tile_e
    if et_len == E:
        et_p = edge_type.astype(jnp.int32).reshape(1, E)
    else:
        et_p = jnp.full((1, et_len), -1, jnp.int32)
        et_p = et_p.at[0, :E].set(edge_type.astype(jnp.int32))

    sums, cnt = pl.pallas_call(
        functools.partial(_segsum_kernel, E, tiles_per_slab,
                          bool(E % tile_e)),
        out_shape=(
            jax.ShapeDtypeStruct((num_slabs, R, c3), f32),
            jax.ShapeDtypeStruct((num_slabs, R, 1), f32),
        ),
        grid=(num_slabs, tiles_per_slab),
        in_specs=[
            pl.BlockSpec((1, tile_e),
                         lambda s, e: (0, s * tiles_per_slab + e)),
            pl.BlockSpec((tile_e, c3),
                         lambda s, e: (jnp.minimum(
                             s * tiles_per_slab + e, last_tile), 0)),
        ],
        out_specs=(
            pl.BlockSpec((None, R, c3), lambda s, e: (s, 0, 0)),
            pl.BlockSpec((None, R, 1), lambda s, e: (s, 0, 0)),
        ),
        compiler_params=pltpu.CompilerParams(
            dimension_semantics=("parallel", "arbitrary"),
            vmem_limit_bytes=64 * 1024 * 1024,
        ),
    )(et_p, c)

    out = pl.pallas_call(
        _finish_kernel,
        out_shape=jax.ShapeDtypeStruct((R, out_size), f32),
    )(sums, cnt, g_initial.astype(f32), wg_t.astype(f32),
      wrel_t.astype(f32), brel.astype(f32))
    return out
